# Initial kernel scaffold; baseline (speedup 1.0000x reference)
import jax
import jax.numpy as jnp
from jax.experimental import pallas as pl


def kernel(x, ln_w, ln_b, w1, b1, w2, b2):
    raise NotImplementedError("write your pallas kernel here")


# speedup vs baseline: 13.4502x; 13.4502x over previous
import functools
import math

import jax
import jax.numpy as jnp
from jax.experimental import pallas as pl
from jax.experimental.pallas import tpu as pltpu

C = 96
H = 48
W = 48
P = H * W
WIN = 13
HALF = WIN // 2
K2 = WIN * WIN
KSEL = 8
NEG_INF = float("-inf")


def _knn_kernel(x_ref, lnw_ref, lnb_ref, w1_ref, b1_ref, w2_ref, b2_ref,
                out_ref, s_ref, wk_ref, agg_ref, den_ref):
    X = x_ref[...]
    n2 = jnp.sum(X * X, axis=0, keepdims=True)
    norm = jnp.maximum(jnp.sqrt(n2), 1e-12)
    XN = X / norm

    p = jax.lax.broadcasted_iota(jnp.int32, (1, P), 1)
    px = p % W
    py = p // W

    def sim_body(k, _):
        dy = k // WIN - HALF
        dx = k % WIN - HALF
        off = dy * W + dx
        sh = pltpu.roll(XN, jnp.mod(-off, P), axis=1)
        s = jnp.sum(XN * sh, axis=0, keepdims=True)
        nx = px + dx
        ny = py + dy
        valid = (nx >= 0) & (nx < W) & (ny >= 0) & (ny < H)
        s_ref[pl.ds(k, 1), :] = jnp.where(valid, s, NEG_INF)
        return 0

    jax.lax.fori_loop(0, K2, sim_body, 0)
    s_ref[K2:, :] = jnp.full((s_ref.shape[0] - K2, P), NEG_INF, jnp.float32)

    wk_ref[...] = s_ref[...]

    def cut_body(_, __):
        w = wk_ref[...]
        m = jnp.max(w, axis=0, keepdims=True)
        wk_ref[...] = jnp.where(w == m, NEG_INF, w)
        return 0

    jax.lax.fori_loop(0, KSEL - 1, cut_body, 0)
    cut = jnp.max(wk_ref[...], axis=0, keepdims=True)

    agg_ref[...] = jnp.zeros((C, P), jnp.float32)
    den_ref[...] = jnp.zeros((1, P), jnp.float32)

    def agg_body(k, _):
        dy = k // WIN - HALF
        dx = k % WIN - HALF
        off = dy * W + dx
        srow = s_ref[pl.ds(k, 1), :]
        wgt = jnp.where(srow >= cut, jnp.exp(srow), 0.0)
        den_ref[...] += wgt
        xsh = pltpu.roll(x_ref[...], jnp.mod(-off, P), axis=1)
        agg_ref[...] += wgt * xsh
        return 0

    jax.lax.fori_loop(0, K2, agg_body, 0)
    agg = agg_ref[...] / den_ref[...]

    mu = jnp.sum(X, axis=0, keepdims=True) * (1.0 / C)
    xc = X - mu
    var = jnp.sum(xc * xc, axis=0, keepdims=True) * (1.0 / C)
    xln = xc / jnp.sqrt(var + 1e-5) * lnw_ref[...] + lnb_ref[...]

    enh = agg + xln

    h = jnp.dot(w1_ref[...], enh, preferred_element_type=jnp.float32)
    h = jnp.maximum(h + b1_ref[...], 0.0)
    ffn = jnp.dot(w2_ref[...], h, preferred_element_type=jnp.float32)
    out_ref[...] = enh + ffn + b2_ref[...]


@jax.jit
def kernel(x, ln_w, ln_b, w1, b1, w2, b2):
    xf = x.reshape(C, P)
    out = pl.pallas_call(
        _knn_kernel,
        out_shape=jax.ShapeDtypeStruct((C, P), jnp.float32),
        scratch_shapes=[
            pltpu.VMEM((176, P), jnp.float32),
            pltpu.VMEM((176, P), jnp.float32),
            pltpu.VMEM((C, P), jnp.float32),
            pltpu.VMEM((1, P), jnp.float32),
        ],
    )(xf, ln_w.reshape(C, 1), ln_b.reshape(C, 1),
      w1, b1.reshape(2 * C, 1), w2, b2.reshape(C, 1))
    return out.reshape(1, C, H, W)
